# Initial kernel scaffold; baseline (speedup 1.0000x reference)
import jax
import jax.numpy as jnp
from jax.experimental import pallas as pl


def kernel(rpn_bbox_deltas, rpn_labels):
    raise NotImplementedError("write your pallas kernel here")


# speedup vs baseline: 5.4047x; 5.4047x over previous
import numpy as np
import jax
import jax.numpy as jnp
from jax import lax
from jax.experimental import pallas as pl
from jax.experimental.pallas import tpu as pltpu

_BASE = np.array([
    [-0.04419417, -0.08838835, 0.04419417, 0.08838835],
    [-0.0625, -0.0625, 0.0625, 0.0625],
    [-0.08838835, -0.04419417, 0.08838835, 0.04419417],
    [-0.08838835, -0.1767767, 0.08838835, 0.1767767],
    [-0.125, -0.125, 0.125, 0.125],
    [-0.1767767, -0.08838835, 0.1767767, 0.08838835],
    [-0.1767767, -0.35355339, 0.1767767, 0.35355339],
    [-0.25, -0.25, 0.25, 0.25],
    [-0.35355339, -0.1767767, 0.35355339, 0.1767767],
], dtype=np.float32)
_PRE = 6000
_POST = 300
_THR = np.float32(0.7)


def _anchor_terms(fm_h, fm_w):
    gy = (np.arange(fm_h, dtype=np.float32) + np.float32(0.5)) / np.float32(fm_h)
    gx = (np.arange(fm_w, dtype=np.float32) + np.float32(0.5)) / np.float32(fm_w)
    gyy, gxx = np.meshgrid(gy, gx, indexing='ij')
    centers = np.stack([gyy, gxx, gyy, gxx], axis=-1).reshape(-1, 1, 4).astype(np.float32)
    anchors = (centers + _BASE[None, :, :]).reshape(-1, 4)
    anchors = np.clip(anchors, np.float32(0.0), np.float32(1.0)).astype(np.float32)
    anc_h = anchors[:, 2] - anchors[:, 0]
    anc_w = anchors[:, 3] - anchors[:, 1]
    anc_cy = anchors[:, 0] + np.float32(0.5) * anc_h
    anc_cx = anchors[:, 1] + np.float32(0.5) * anc_w
    return anc_h, anc_w, anc_cy, anc_cx


def _nms_kernel(scores_ref, dy_ref, dx_ref, dh_ref, dw_ref,
                ah_ref, aw_ref, acy_ref, acx_ref,
                oy1_ref, ox1_ref, oy2_ref, ox2_ref,
                y1_s, x1_s, y2_s, x2_s, area_s, ws_s):
    b, n = scores_ref.shape

    ah = ah_ref[...]
    aw = aw_ref[...]
    bb_h = jnp.exp(dh_ref[...] * np.float32(0.2)) * ah
    bb_w = jnp.exp(dw_ref[...] * np.float32(0.2)) * aw
    bb_cy = dy_ref[...] * np.float32(0.1) * ah + acy_ref[...]
    bb_cx = dx_ref[...] * np.float32(0.1) * aw + acx_ref[...]
    y1 = bb_cy - np.float32(0.5) * bb_h
    x1 = bb_cx - np.float32(0.5) * bb_w
    y2 = y1 + bb_h
    x2 = x1 + bb_w
    y1_s[...] = y1
    x1_s[...] = x1
    y2_s[...] = y2
    x2_s[...] = x2
    area_s[...] = jnp.maximum(y2 - y1, 0.0) * jnp.maximum(x2 - x1, 0.0)

    scores = scores_ref[...]
    keys = lax.bitcast_convert_type(scores, jnp.int32)
    iota = lax.broadcasted_iota(jnp.int32, (b, n), 1)
    oy1_ref[...] = jnp.zeros_like(oy1_ref)
    ox1_ref[...] = jnp.zeros_like(ox1_ref)
    oy2_ref[...] = jnp.zeros_like(oy2_ref)
    ox2_ref[...] = jnp.zeros_like(ox2_ref)
    iota_p = lax.broadcasted_iota(jnp.int32, oy1_ref.shape, 1)

    def thr_body(_, state):
        lo, hi = state
        mid = (lo + hi + 1) >> 1
        cnt = jnp.sum((keys >= mid).astype(jnp.int32), axis=1, keepdims=True)
        ok = cnt >= _PRE
        lo = jnp.where(ok, mid, lo)
        hi = jnp.where(ok, hi, mid - 1)
        return lo, hi

    lo0 = jnp.zeros((b, 1), jnp.int32)
    hi0 = jnp.full((b, 1), 0x3F800000, jnp.int32)
    tau, _ = lax.fori_loop(0, 31, thr_body, (lo0, hi0))

    c_gt = jnp.sum((keys > tau).astype(jnp.int32), axis=1, keepdims=True)
    r = _PRE - c_gt
    eq = (keys == tau)

    def idx_body(_, state):
        lo, hi = state
        mid = (lo + hi) >> 1
        cnt = jnp.sum((eq & (iota <= mid)).astype(jnp.int32), axis=1, keepdims=True)
        ok = cnt >= r
        hi = jnp.where(ok, mid, hi)
        lo = jnp.where(ok, lo, mid + 1)
        return lo, hi

    ilo0 = jnp.zeros((b, 1), jnp.int32)
    ihi0 = jnp.full((b, 1), n - 1, jnp.int32)
    _, icut = lax.fori_loop(0, 15, idx_body, (ilo0, ihi0))
    icut = jnp.where(r > 0, icut, -1)

    active = (keys > tau) | (eq & (iota <= icut))
    ws_s[...] = jnp.where(active, scores, np.float32(-1.0))

    def nms_body(i, _):
        ws = ws_s[...]
        m = jnp.max(ws, axis=1, keepdims=True)
        pos = jnp.min(jnp.where(ws == m, iota, n), axis=1, keepdims=True)
        wmask = (iota == pos)
        wz = wmask.astype(jnp.float32)
        by1 = jnp.sum(y1_s[...] * wz, axis=1, keepdims=True)
        bx1 = jnp.sum(x1_s[...] * wz, axis=1, keepdims=True)
        by2 = jnp.sum(y2_s[...] * wz, axis=1, keepdims=True)
        bx2 = jnp.sum(x2_s[...] * wz, axis=1, keepdims=True)
        valid = (m > 0.0).astype(jnp.float32)
        hot = (iota_p == i).astype(jnp.float32)
        oy1_ref[...] += hot * (by1 * valid)
        ox1_ref[...] += hot * (bx1 * valid)
        oy2_ref[...] += hot * (by2 * valid)
        ox2_ref[...] += hot * (bx2 * valid)
        yy1 = jnp.maximum(by1, y1_s[...])
        xx1 = jnp.maximum(bx1, x1_s[...])
        yy2 = jnp.minimum(by2, y2_s[...])
        xx2 = jnp.minimum(bx2, x2_s[...])
        inter = jnp.maximum(yy2 - yy1, 0.0) * jnp.maximum(xx2 - xx1, 0.0)
        barea = jnp.maximum(by2 - by1, 0.0) * jnp.maximum(bx2 - bx1, 0.0)
        iou = inter / jnp.maximum(area_s[...] + barea - inter, np.float32(1e-8))
        ws = jnp.where(iou > _THR, np.float32(-1.0), ws)
        ws = jnp.where(wmask, np.float32(-1.0), ws)
        ws_s[...] = ws
        return 0

    lax.fori_loop(0, _POST, nms_body, 0)


def kernel(rpn_bbox_deltas, rpn_labels):
    b = rpn_bbox_deltas.shape[0]
    fm_h, fm_w = rpn_labels.shape[1], rpn_labels.shape[2]
    n = fm_h * fm_w * 9
    deltas = rpn_bbox_deltas.reshape(b, n, 4)
    scores = rpn_labels.reshape(b, n)
    dy = deltas[:, :, 0]
    dx = deltas[:, :, 1]
    dh = deltas[:, :, 2]
    dw = deltas[:, :, 3]
    anc_h, anc_w, anc_cy, anc_cx = _anchor_terms(fm_h, fm_w)
    ah = jnp.asarray(anc_h).reshape(1, n)
    aw = jnp.asarray(anc_w).reshape(1, n)
    acy = jnp.asarray(anc_cy).reshape(1, n)
    acx = jnp.asarray(anc_cx).reshape(1, n)

    f32 = jnp.float32
    outs = [jax.ShapeDtypeStruct((b, _POST), f32)] * 4
    scratch = [pltpu.VMEM((b, n), f32)] * 6
    oy1, ox1, oy2, ox2 = pl.pallas_call(
        _nms_kernel,
        out_shape=outs,
        scratch_shapes=scratch,
    )(scores, dy, dx, dh, dw, ah, aw, acy, acx)
    out = jnp.stack([oy1, ox1, oy2, ox2], axis=-1)
    return lax.stop_gradient(out)
